# Initial kernel scaffold; baseline (speedup 1.0000x reference)
import jax
import jax.numpy as jnp
from jax.experimental import pallas as pl


def kernel(x, weight):
    raise NotImplementedError("write your pallas kernel here")


# speedup vs baseline: 1.8311x; 1.8311x over previous
import functools

import jax
import jax.numpy as jnp
from jax import lax
from jax.experimental import pallas as pl
from jax.experimental.pallas import tpu as pltpu
from jax.experimental.pallas import tpu_sc as plsc

D_MODEL = 64
NUM_CORES = 2
NUM_SUBCORES = 16
NUM_WORKERS = NUM_CORES * NUM_SUBCORES
CHUNK = 512


@functools.lru_cache(maxsize=None)
def _make_lookup(B: int):
    assert B % (NUM_WORKERS * CHUNK) == 0
    b_per_w = B // NUM_WORKERS
    n_chunks = b_per_w // CHUNK
    mesh = plsc.VectorSubcoreMesh(
        core_axis_name="c", subcore_axis_name="s",
        num_cores=NUM_CORES, num_subcores=NUM_SUBCORES)

    @functools.partial(
        pl.kernel,
        out_type=jax.ShapeDtypeStruct((B, D_MODEL), jnp.float32),
        mesh=mesh,
        scratch_types=[
            pltpu.VMEM((b_per_w,), jnp.int32),
            pltpu.VMEM((CHUNK, D_MODEL), jnp.float32),
            pltpu.SemaphoreType.DMA,
        ],
        compiler_params=pltpu.CompilerParams(use_tc_tiling_on_sc=False),
    )
    def lookup(table_hbm, idx_hbm, out_hbm, idx_v, rows_v, sem):
        wid = lax.axis_index("s") * NUM_CORES + lax.axis_index("c")
        base = wid * b_per_w
        pltpu.sync_copy(idx_hbm.at[pl.ds(base, b_per_w)], idx_v)

        def body(i, carry):
            off = i * CHUNK
            pltpu.async_copy(
                table_hbm.at[idx_v.at[pl.ds(off, CHUNK)]], rows_v, sem
            ).wait()
            pltpu.sync_copy(rows_v, out_hbm.at[pl.ds(base + off, CHUNK)])
            return carry

        lax.fori_loop(0, n_chunks, body, 0)

    return lookup


@jax.jit
def kernel(x, weight):
    B = x.shape[0] * x.shape[1]
    flat = x.reshape(B).astype(jnp.int32)
    out = _make_lookup(B)(weight, flat)
    return out.reshape(x.shape[0], x.shape[1], D_MODEL)


# speedup vs baseline: 1.8692x; 1.0208x over previous
import functools

import jax
import jax.numpy as jnp
from jax import lax
from jax.experimental import pallas as pl
from jax.experimental.pallas import tpu as pltpu
from jax.experimental.pallas import tpu_sc as plsc

D_MODEL = 64
NUM_CORES = 2
NUM_SUBCORES = 16
NUM_WORKERS = NUM_CORES * NUM_SUBCORES
CHUNK = 256
NBUF = 4


@functools.lru_cache(maxsize=None)
def _make_lookup(B: int):
    assert B % (NUM_WORKERS * CHUNK * NBUF) == 0
    b_per_w = B // NUM_WORKERS
    n_chunks = b_per_w // CHUNK
    n_rounds = n_chunks // NBUF
    mesh = plsc.VectorSubcoreMesh(
        core_axis_name="c", subcore_axis_name="s",
        num_cores=NUM_CORES, num_subcores=NUM_SUBCORES)

    @functools.partial(
        pl.kernel,
        out_type=jax.ShapeDtypeStruct((B, D_MODEL), jnp.float32),
        mesh=mesh,
        scratch_types=[
            pltpu.VMEM((b_per_w,), jnp.int32),
            pltpu.VMEM((NBUF, CHUNK, D_MODEL), jnp.float32),
        ] + [pltpu.SemaphoreType.DMA] * (2 * NBUF),
        compiler_params=pltpu.CompilerParams(use_tc_tiling_on_sc=False),
    )
    def lookup(table_hbm, idx_hbm, out_hbm, idx_v, rows_v, *sems):
        sem_g = sems[:NBUF]
        sem_o = sems[NBUF:]
        wid = lax.axis_index("s") * NUM_CORES + lax.axis_index("c")
        base = wid * b_per_w
        pltpu.sync_copy(idx_hbm.at[pl.ds(base, b_per_w)], idx_v)

        def gather_desc(i, b):
            return pltpu.make_async_copy(
                table_hbm.at[idx_v.at[pl.ds(i * CHUNK, CHUNK)]],
                rows_v.at[b], sem_g[b])

        def out_desc(i, b):
            return pltpu.make_async_copy(
                rows_v.at[b], out_hbm.at[pl.ds(base + i * CHUNK, CHUNK)],
                sem_o[b])

        for b in range(NBUF):
            gather_desc(b, b).start()

        def body(r, carry):
            for b in range(NBUF):
                i = r * NBUF + b
                gather_desc(i, b).wait()
                out_desc(i, b).start()
            for b in range(NBUF):
                i = r * NBUF + b
                out_desc(i, b).wait()
                gather_desc(i + NBUF, b).start()
            return carry

        lax.fori_loop(0, n_rounds - 1, body, 0)

        r_last = n_rounds - 1
        for b in range(NBUF):
            i = r_last * NBUF + b
            gather_desc(i, b).wait()
            out_desc(i, b).start()
        for b in range(NBUF):
            out_desc(r_last * NBUF + b, b).wait()

    return lookup


@jax.jit
def kernel(x, weight):
    B = x.shape[0] * x.shape[1]
    flat = x.reshape(B).astype(jnp.int32)
    out = _make_lookup(B)(weight, flat)
    return out.reshape(x.shape[0], x.shape[1], D_MODEL)
